# Initial kernel scaffold; baseline (speedup 1.0000x reference)
import jax
import jax.numpy as jnp
from jax.experimental import pallas as pl


def kernel(x, batch, edge_index, W1, b1, W2, b2, W3, b3, W4, b4):
    raise NotImplementedError("write your pallas kernel here")


# speedup vs baseline: 41.2428x; 41.2428x over previous
import functools

import jax
import jax.numpy as jnp
from jax import lax
from jax.experimental import pallas as pl
from jax.experimental.pallas import tpu as pltpu
from jax.experimental.pallas import tpu_sc as plsc

_N = 10000
_G = 256
_K = 16
_NW = 32
_PER_W = 320
_NPAD = _NW * _PER_W
_NK = _NPAD * _K
_EB = 2048
_NBLK = _NK // _EB
_STARTS_PAD = 512
_XPAD = _NPAD + 16
_INIT = float(3e38)


def _leaky(t):
    return jnp.where(t > 0, t, 0.01 * t)


def _rbf(t):
    y = lax.bitcast_convert_type(t, jnp.int32)
    r = (y + 0x7FFF + ((y >> 16) & 1)) & jnp.int32(-65536)
    return lax.bitcast_convert_type(r, jnp.float32)


def _rbf_sc(v):
    y = plsc.bitcast(v, jnp.int32)
    r = (y + 0x7FFF + ((y >> 16) & 1)) & jnp.int32(-65536)
    return plsc.bitcast(r, jnp.float32)


def _merge16(ak, ai, bk, bi):
    rbk = jnp.flip(bk, 0)
    rbi = jnp.flip(bi, 0)
    cond = ak <= rbk
    nk = jnp.where(cond, ak, rbk)
    ni = jnp.where(cond, ai, rbi)
    sk, si = plsc.sort_key_val(nk, ni)
    return sk, si


def _knn_gather_body(xt_hbm, batch_hbm, starts_hbm, out_hbm,
                     x0, x1, x2, x3, batch_v, starts_v, obuf):
    cid = lax.axis_index("c")
    sid = lax.axis_index("s")
    w = sid * 2 + cid
    pltpu.sync_copy(xt_hbm.at[0], x0)
    pltpu.sync_copy(xt_hbm.at[1], x1)
    pltpu.sync_copy(xt_hbm.at[2], x2)
    pltpu.sync_copy(xt_hbm.at[3], x3)
    pltpu.sync_copy(batch_hbm, batch_v)
    pltpu.sync_copy(starts_hbm, starts_v)
    iot = lax.iota(jnp.int32, 16)
    iotf = iot.astype(jnp.float32)

    def node_body(n, carry):
        i = w * _PER_W + n

        @pl.when(i < _N)
        def _():
            g = batch_v[pl.ds(i, 16)][0]
            sv = starts_v[pl.ds(g, 16)]
            s = sv[0]
            e = sv[1]
            xi0 = x0[pl.ds(i, 16)][0]
            xi1 = x1[pl.ds(i, 16)][0]
            xi2 = x2[pl.ds(i, 16)][0]
            xi3 = x3[pl.ds(i, 16)][0]
            nch = (e - s + 15) // 16

            def chunk_body(t, bc):
                bk, bi = bc
                c = s + t * 16
                idx = c + iot
                d0 = x0[pl.ds(c, 16)] - xi0
                d1 = x1[pl.ds(c, 16)] - xi1
                d2c = x2[pl.ds(c, 16)] - xi2
                dist = d0 * d0 + d1 * d1 + d2c * d2c
                valid = (idx != i) & (idx < e)
                key = jnp.where(valid, dist, _INIT)
                ck, ci = plsc.sort_key_val(key, idx)
                return _merge16(bk, bi, ck, ci)

            bk0 = jnp.full((16,), _INIT, jnp.float32)
            bi0 = jnp.zeros((16,), jnp.int32)
            bk, bi = lax.fori_loop(0, nch, chunk_body, (bk0, bi0))
            fill = jnp.where(iot < s, iot,
                             jnp.where(iot == s, i, e + iot - s - 1))
            fkey = jnp.float32(1e30) * (1.0 + fill.astype(jnp.float32)
                                        * jnp.float32(2.0 ** -10))
            fk, fi = plsc.sort_key_val(fkey, fill)
            bk, bi = _merge16(bk, bi, fk, fi)

            base = n * 16
            for f, (xf, xif) in enumerate(
                    ((x0, xi0), (x1, xi1), (x2, xi2), (x3, xi3))):
                xj = plsc.load_gather(xf, [bi])
                obuf[f, pl.ds(base, 16)] = jnp.broadcast_to(xif, (16,))
                obuf[4 + f, pl.ds(base, 16)] = xj - xif
        return carry

    lax.fori_loop(0, _PER_W, node_body, 0)
    for f in range(8):
        pltpu.sync_copy(obuf.at[f], out_hbm.at[f, w])


def _edgeconv_body(w1_ref, b1_ref, w2_ref, b2_ref, w3_ref, b3_ref,
                   planes_ref, h_ref):
    ch = [_rbf(planes_ref[f]) for f in range(8)]
    m1 = []
    for o in range(15):
        acc = ch[0] * w1_ref[0, o]
        for f in range(1, 8):
            acc = acc + ch[f] * w1_ref[f, o]
        m1.append(_rbf(_leaky(acc + b1_ref[o])))
    hq = []
    for o in range(15):
        acc = m1[0] * w2_ref[0, o]
        for p in range(1, 15):
            acc = acc + m1[p] * w2_ref[p, o]
        m2 = _leaky(acc + b2_ref[o])
        hq.append(_rbf(jnp.sum(m2.reshape(16, 8, 16), axis=2)))
    hpre = hq[0] * w3_ref[0, 0]
    for o in range(1, 15):
        hpre = hpre + hq[o] * w3_ref[o, 0]
    h_ref[...] = _leaky(hpre + b3_ref[0])


def _pool_body(h_hbm, starts_hbm, w4b_hbm, out_hbm,
               h_v, starts_v, w4_v, out_v):
    cid = lax.axis_index("c")
    sid = lax.axis_index("s")
    w = sid * 2 + cid

    @pl.when(w < 16)
    def _():
        pltpu.sync_copy(h_hbm, h_v)
        pltpu.sync_copy(starts_hbm, starts_v)
        pltpu.sync_copy(w4b_hbm, w4_v)
        iot = lax.iota(jnp.int32, 16)
        a_vec = jnp.zeros((16,), jnp.float32)
        bm_vec = jnp.zeros((16,), jnp.float32)
        sm_vec = jnp.zeros((16,), jnp.float32)
        for gi in range(16):
            g = w * 16 + gi
            sv = starts_v[pl.ds(g, 16)]
            s = sv[0]
            e = sv[1]
            nch = (e - s + 15) // 16

            def chunk_body(t, carry):
                vmax, vmin, vsum = carry
                c = s + t * 16
                v = h_v[pl.ds(c, 16)]
                m = (c + iot) < e
                vmax = jnp.maximum(vmax, jnp.where(m, v, -_INIT))
                vmin = jnp.minimum(vmin, jnp.where(m, v, _INIT))
                vsum = vsum + jnp.where(m, v, 0.0)
                return vmax, vmin, vsum

            vmax, vmin, vsum = lax.fori_loop(
                0, nch, chunk_body,
                (jnp.full((16,), -_INIT, jnp.float32),
                 jnp.full((16,), _INIT, jnp.float32),
                 jnp.zeros((16,), jnp.float32)))
            lane = iot == gi
            a_vec = jnp.where(lane, jnp.max(vmax), a_vec)
            bm_vec = jnp.where(lane, jnp.min(vmin), bm_vec)
            sm_vec = jnp.where(lane, jnp.sum(vsum), sm_vec)
        s_vec = plsc.load_gather(starts_v, [w * 16 + iot])
        e_vec = plsc.load_gather(starts_v, [w * 16 + 1 + iot])
        cnt = (e_vec - s_vec).astype(jnp.float32)
        pos = cnt > 0
        a = _rbf_sc(jnp.where(pos, a_vec, 0.0))
        bm = _rbf_sc(jnp.where(pos, bm_vec, 0.0))
        sm = _rbf_sc(sm_vec)
        d = _rbf_sc(sm_vec / jnp.maximum(cnt, 1.0))
        w4 = w4_v[...]
        out_v[...] = (a * w4[0] + bm * w4[1] + sm * w4[2]
                      + d * w4[3] + w4[4])
        pltpu.sync_copy(out_v, out_hbm.at[pl.ds(w * 16, 16)])


def _build_sc_kernels():
    mesh = plsc.VectorSubcoreMesh(core_axis_name="c", subcore_axis_name="s",
                                  num_cores=2, num_subcores=16)
    cparams = pltpu.CompilerParams(needs_layout_passes=False)
    knn = pl.kernel(
        _knn_gather_body,
        out_type=jax.ShapeDtypeStruct((8, _NW, _PER_W * _K), jnp.float32),
        mesh=mesh,
        compiler_params=cparams,
        scratch_types=[
            pltpu.VMEM((_XPAD,), jnp.float32),
            pltpu.VMEM((_XPAD,), jnp.float32),
            pltpu.VMEM((_XPAD,), jnp.float32),
            pltpu.VMEM((_XPAD,), jnp.float32),
            pltpu.VMEM((_XPAD,), jnp.int32),
            pltpu.VMEM((_STARTS_PAD,), jnp.int32),
            pltpu.VMEM((8, _PER_W * _K), jnp.float32),
        ],
    )
    pool = pl.kernel(
        _pool_body,
        out_type=jax.ShapeDtypeStruct((_G,), jnp.float32),
        mesh=mesh,
        compiler_params=cparams,
        scratch_types=[
            pltpu.VMEM((_NPAD,), jnp.float32),
            pltpu.VMEM((_STARTS_PAD,), jnp.int32),
            pltpu.VMEM((16,), jnp.float32),
            pltpu.VMEM((16,), jnp.float32),
        ],
    )
    return knn, pool


def kernel(x, batch, edge_index, W1, b1, W2, b2, W3, b3, W4, b4):
    del edge_index
    x = x.astype(jnp.float32)
    batch = batch.astype(jnp.int32)

    xt = jnp.zeros((4, _XPAD), jnp.float32).at[:, :_N].set(x.T)
    batch_p = jnp.zeros((_XPAD,), jnp.int32).at[:_N].set(batch)
    starts = jnp.searchsorted(batch, jnp.arange(_G + 1, dtype=jnp.int32)
                              ).astype(jnp.int32)
    starts = jnp.concatenate(
        [starts, jnp.full((_STARTS_PAD - _G - 1,), _N, jnp.int32)])
    w1r = _rbf(W1.astype(jnp.float32))
    w2r = _rbf(W2.astype(jnp.float32))
    w3r = _rbf(W3.astype(jnp.float32))
    w4b = (jnp.zeros((16,), jnp.float32)
           .at[:4].set(_rbf(W4[:, 0].astype(jnp.float32)))
           .at[4].set(b4[0]))

    knn, pool = _build_sc_kernels()
    planes = knn(xt, batch_p, starts)
    planes = planes.reshape(8, _NK // 128, 128)

    h = pl.pallas_call(
        _edgeconv_body,
        grid=(_NBLK,),
        in_specs=[
            pl.BlockSpec(memory_space=pltpu.SMEM),
            pl.BlockSpec(memory_space=pltpu.SMEM),
            pl.BlockSpec(memory_space=pltpu.SMEM),
            pl.BlockSpec(memory_space=pltpu.SMEM),
            pl.BlockSpec(memory_space=pltpu.SMEM),
            pl.BlockSpec(memory_space=pltpu.SMEM),
            pl.BlockSpec((8, 16, 128), lambda j: (0, j, 0)),
        ],
        out_specs=pl.BlockSpec((16, 8), lambda j: (j, 0)),
        out_shape=jax.ShapeDtypeStruct((_NK // 128, 8), jnp.float32),
    )(w1r, b1, w2r, b2, w3r, b3, planes)

    out = pool(h.reshape(_NPAD), starts, w4b)
    return out.reshape(_G, 1)


# speedup vs baseline: 59.5149x; 1.4430x over previous
import functools

import jax
import jax.numpy as jnp
from jax import lax
from jax.experimental import pallas as pl
from jax.experimental.pallas import tpu as pltpu
from jax.experimental.pallas import tpu_sc as plsc

_N = 10000
_G = 256
_K = 16
_NW = 32
_PER_W = 320
_NPAD = _NW * _PER_W
_NK = _NPAD * _K
_TCR = 64
_NBLK = _NK // (_TCR * 128)
_STARTS_PAD = 512
_XPAD = _NPAD + 16
_INIT = float(3e38)


def _leaky(t):
    return jnp.where(t > 0, t, 0.01 * t)


def _rbf(t):
    y = lax.bitcast_convert_type(t, jnp.int32)
    r = (y + 0x7FFF + ((y >> 16) & 1)) & jnp.int32(-65536)
    return lax.bitcast_convert_type(r, jnp.float32)


def _rbf_sc(v):
    y = plsc.bitcast(v, jnp.int32)
    r = (y + 0x7FFF + ((y >> 16) & 1)) & jnp.int32(-65536)
    return plsc.bitcast(r, jnp.float32)


def _merge16(ak, ai, bk, bi):
    rbk = jnp.flip(bk, 0)
    rbi = jnp.flip(bi, 0)
    cond = ak <= rbk
    nk = jnp.where(cond, ak, rbk)
    ni = jnp.where(cond, ai, rbi)
    sk, si = plsc.sort_key_val(nk, ni)
    return sk, si


def _knn_gather_body(xt_hbm, batch_hbm, starts_hbm, out_hbm,
                     x0, x1, x2, x3, batch_v, starts_v, obuf):
    cid = lax.axis_index("c")
    sid = lax.axis_index("s")
    w = sid * 2 + cid
    pltpu.sync_copy(xt_hbm.at[0], x0)
    pltpu.sync_copy(xt_hbm.at[1], x1)
    pltpu.sync_copy(xt_hbm.at[2], x2)
    pltpu.sync_copy(xt_hbm.at[3], x3)
    pltpu.sync_copy(batch_hbm, batch_v)
    pltpu.sync_copy(starts_hbm, starts_v)
    iot = lax.iota(jnp.int32, 16)
    iotf = iot.astype(jnp.float32)

    def node_body(n, carry):
        i = w * _PER_W + n

        @pl.when(i < _N)
        def _():
            g = batch_v[pl.ds(i, 16)][0]
            sv = starts_v[pl.ds(g, 16)]
            s = sv[0]
            e = sv[1]
            xi0 = x0[pl.ds(i, 16)][0]
            xi1 = x1[pl.ds(i, 16)][0]
            xi2 = x2[pl.ds(i, 16)][0]
            xi3 = x3[pl.ds(i, 16)][0]
            nch = (e - s + 15) // 16

            def chunk_keys(c):
                idx = c + iot
                d0 = x0[pl.ds(c, 16)] - xi0
                d1 = x1[pl.ds(c, 16)] - xi1
                d2c = x2[pl.ds(c, 16)] - xi2
                dist = d0 * d0 + d1 * d1 + d2c * d2c
                valid = (idx != i) & (idx < e)
                key = jnp.where(valid, dist, _INIT)
                return plsc.sort_key_val(key, idx)

            def chunk_body(t, bc):
                bk, bi = bc
                ck, ci = chunk_keys(s + t * 16)
                return _merge16(bk, bi, ck, ci)

            ck0, ci0 = chunk_keys(s)
            bk, bi = lax.fori_loop(1, nch, chunk_body, (ck0, ci0))

            def _with_fill():
                fill = jnp.where(iot < s, iot,
                                 jnp.where(iot == s, i, e + iot - s - 1))
                fkey = jnp.float32(1e30) * (1.0 + fill.astype(jnp.float32)
                                            * jnp.float32(2.0 ** -10))
                fk, fi = plsc.sort_key_val(fkey, fill)
                return _merge16(bk, bi, fk, fi)

            bk, bi = lax.cond(e - s < 17, _with_fill, lambda: (bk, bi))

            base = n * 16
            for f, (xf, xif) in enumerate(
                    ((x0, xi0), (x1, xi1), (x2, xi2), (x3, xi3))):
                xj = plsc.load_gather(xf, [bi])
                obuf[f, pl.ds(base, 16)] = jnp.broadcast_to(xif, (16,))
                obuf[4 + f, pl.ds(base, 16)] = xj - xif
        return carry

    lax.fori_loop(0, _PER_W, node_body, 0)
    for f in range(8):
        pltpu.sync_copy(obuf.at[f], out_hbm.at[f, w])


def _edgeconv_body(w1_ref, b1_ref, w2_ref, b2_ref, w3_ref, b3_ref,
                   planes_ref, h_ref):
    ch = [_rbf(planes_ref[f]) for f in range(8)]
    m1 = []
    for o in range(15):
        acc = ch[0] * w1_ref[0, o]
        for f in range(1, 8):
            acc = acc + ch[f] * w1_ref[f, o]
        m1.append(_rbf(_leaky(acc + b1_ref[o])))
    hpre = None
    for o in range(15):
        acc = m1[0] * w2_ref[0, o]
        for p in range(1, 15):
            acc = acc + m1[p] * w2_ref[p, o]
        m2 = _leaky(acc + b2_ref[o])
        for sh in (1, 2, 4, 8):
            m2 = m2 + pltpu.roll(m2, 128 - sh, 1)
        hq = _rbf(m2)
        contrib = hq * w3_ref[o, 0]
        hpre = contrib if hpre is None else hpre + contrib
    lane = lax.broadcasted_iota(jnp.int32, (_TCR, 128), 1)
    hsel = jnp.where(lane % 16 == 0, hpre, 0.0)
    h_ref[...] = _leaky(jnp.sum(hsel.reshape(_TCR, 8, 16), axis=2)
                        + b3_ref[0])


def _pool_body(h_hbm, starts_hbm, w4b_hbm, out_hbm,
               h_v, starts_v, w4_v, out_v):
    cid = lax.axis_index("c")
    sid = lax.axis_index("s")
    w = sid * 2 + cid

    @pl.when(w < 16)
    def _():
        pltpu.sync_copy(h_hbm, h_v)
        pltpu.sync_copy(starts_hbm, starts_v)
        pltpu.sync_copy(w4b_hbm, w4_v)
        iot = lax.iota(jnp.int32, 16)
        a_vec = jnp.zeros((16,), jnp.float32)
        bm_vec = jnp.zeros((16,), jnp.float32)
        sm_vec = jnp.zeros((16,), jnp.float32)
        for gi in range(16):
            g = w * 16 + gi
            sv = starts_v[pl.ds(g, 16)]
            s = sv[0]
            e = sv[1]
            nch = (e - s + 15) // 16

            def chunk_body(t, carry):
                vmax, vmin, vsum = carry
                c = s + t * 16
                v = h_v[pl.ds(c, 16)]
                m = (c + iot) < e
                vmax = jnp.maximum(vmax, jnp.where(m, v, -_INIT))
                vmin = jnp.minimum(vmin, jnp.where(m, v, _INIT))
                vsum = vsum + jnp.where(m, v, 0.0)
                return vmax, vmin, vsum

            vmax, vmin, vsum = lax.fori_loop(
                0, nch, chunk_body,
                (jnp.full((16,), -_INIT, jnp.float32),
                 jnp.full((16,), _INIT, jnp.float32),
                 jnp.zeros((16,), jnp.float32)))
            lane = iot == gi
            a_vec = jnp.where(lane, jnp.max(vmax), a_vec)
            bm_vec = jnp.where(lane, jnp.min(vmin), bm_vec)
            sm_vec = jnp.where(lane, jnp.sum(vsum), sm_vec)
        s_vec = plsc.load_gather(starts_v, [w * 16 + iot])
        e_vec = plsc.load_gather(starts_v, [w * 16 + 1 + iot])
        cnt = (e_vec - s_vec).astype(jnp.float32)
        pos = cnt > 0
        a = _rbf_sc(jnp.where(pos, a_vec, 0.0))
        bm = _rbf_sc(jnp.where(pos, bm_vec, 0.0))
        sm = _rbf_sc(sm_vec)
        d = _rbf_sc(sm_vec / jnp.maximum(cnt, 1.0))
        w4 = w4_v[...]
        out_v[...] = (a * w4[0] + bm * w4[1] + sm * w4[2]
                      + d * w4[3] + w4[4])
        pltpu.sync_copy(out_v, out_hbm.at[pl.ds(w * 16, 16)])


def _build_sc_kernels():
    mesh = plsc.VectorSubcoreMesh(core_axis_name="c", subcore_axis_name="s",
                                  num_cores=2, num_subcores=16)
    cparams = pltpu.CompilerParams(needs_layout_passes=False)
    knn = pl.kernel(
        _knn_gather_body,
        out_type=jax.ShapeDtypeStruct((8, _NW, _PER_W * _K), jnp.float32),
        mesh=mesh,
        compiler_params=cparams,
        scratch_types=[
            pltpu.VMEM((_XPAD,), jnp.float32),
            pltpu.VMEM((_XPAD,), jnp.float32),
            pltpu.VMEM((_XPAD,), jnp.float32),
            pltpu.VMEM((_XPAD,), jnp.float32),
            pltpu.VMEM((_XPAD,), jnp.int32),
            pltpu.VMEM((_STARTS_PAD,), jnp.int32),
            pltpu.VMEM((8, _PER_W * _K), jnp.float32),
        ],
    )
    pool = pl.kernel(
        _pool_body,
        out_type=jax.ShapeDtypeStruct((_G,), jnp.float32),
        mesh=mesh,
        compiler_params=cparams,
        scratch_types=[
            pltpu.VMEM((_NPAD,), jnp.float32),
            pltpu.VMEM((_STARTS_PAD,), jnp.int32),
            pltpu.VMEM((16,), jnp.float32),
            pltpu.VMEM((16,), jnp.float32),
        ],
    )
    return knn, pool


def kernel(x, batch, edge_index, W1, b1, W2, b2, W3, b3, W4, b4):
    del edge_index
    x = x.astype(jnp.float32)
    batch = batch.astype(jnp.int32)

    xt = jnp.zeros((4, _XPAD), jnp.float32).at[:, :_N].set(x.T)
    batch_p = jnp.zeros((_XPAD,), jnp.int32).at[:_N].set(batch)
    starts = jnp.searchsorted(batch, jnp.arange(_G + 1, dtype=jnp.int32)
                              ).astype(jnp.int32)
    starts = jnp.concatenate(
        [starts, jnp.full((_STARTS_PAD - _G - 1,), _N, jnp.int32)])
    w1r = _rbf(W1.astype(jnp.float32))
    w2r = _rbf(W2.astype(jnp.float32))
    w3r = _rbf(W3.astype(jnp.float32))
    w4b = (jnp.zeros((16,), jnp.float32)
           .at[:4].set(_rbf(W4[:, 0].astype(jnp.float32)))
           .at[4].set(b4[0]))

    knn, pool = _build_sc_kernels()
    planes = knn(xt, batch_p, starts)
    planes = planes.reshape(8, _NK // 128, 128)

    h = pl.pallas_call(
        _edgeconv_body,
        grid=(_NBLK,),
        in_specs=[
            pl.BlockSpec(memory_space=pltpu.SMEM),
            pl.BlockSpec(memory_space=pltpu.SMEM),
            pl.BlockSpec(memory_space=pltpu.SMEM),
            pl.BlockSpec(memory_space=pltpu.SMEM),
            pl.BlockSpec(memory_space=pltpu.SMEM),
            pl.BlockSpec(memory_space=pltpu.SMEM),
            pl.BlockSpec((8, _TCR, 128), lambda j: (0, j, 0)),
        ],
        out_specs=pl.BlockSpec((_TCR, 8), lambda j: (j, 0)),
        out_shape=jax.ShapeDtypeStruct((_NK // 128, 8), jnp.float32),
    )(w1r, b1, w2r, b2, w3r, b3, planes)

    out = pool(h.reshape(_NPAD), starts, w4b)
    return out.reshape(_G, 1)


# speedup vs baseline: 59.5660x; 1.0009x over previous
import jax
import jax.numpy as jnp
from jax import lax
from jax.experimental import pallas as pl
from jax.experimental.pallas import tpu as pltpu
from jax.experimental.pallas import tpu_sc as plsc

_N = 10000
_G = 256
_K = 16
_NW = 32
_PER_W = 320
_NPAD = _NW * _PER_W
_NK = _NPAD * _K
_TCR = 64
_NBLK = _NK // (_TCR * 128)
_STARTS_PAD = 512
_XPAD = _NPAD + 16
_INIT = float(3e38)


def _leaky(t):
    return jnp.where(t > 0, t, 0.01 * t)


def _rbf(t):
    y = lax.bitcast_convert_type(t, jnp.int32)
    r = (y + 0x7FFF + ((y >> 16) & 1)) & jnp.int32(-65536)
    return lax.bitcast_convert_type(r, jnp.float32)


def _rbf_sc(v):
    y = plsc.bitcast(v, jnp.int32)
    r = (y + 0x7FFF + ((y >> 16) & 1)) & jnp.int32(-65536)
    return plsc.bitcast(r, jnp.float32)


def _merge16(ak, ai, bk, bi):
    rbk = jnp.flip(bk, 0)
    rbi = jnp.flip(bi, 0)
    cond = ak <= rbk
    nk = jnp.where(cond, ak, rbk)
    ni = jnp.where(cond, ai, rbi)
    sk, si = plsc.sort_key_val(nk, ni)
    return sk, si


def _knn_gather_body(xt_hbm, batch_hbm, starts_hbm, out_hbm,
                     x0, x1, x2, x3, batch_v, starts_v, obuf):
    cid = lax.axis_index("c")
    sid = lax.axis_index("s")
    w = sid * 2 + cid
    pltpu.sync_copy(xt_hbm.at[0], x0)
    pltpu.sync_copy(xt_hbm.at[1], x1)
    pltpu.sync_copy(xt_hbm.at[2], x2)
    pltpu.sync_copy(xt_hbm.at[3], x3)
    pltpu.sync_copy(batch_hbm, batch_v)
    pltpu.sync_copy(starts_hbm, starts_v)
    iot = lax.iota(jnp.int32, 16)

    @plsc.parallel_loop(0, _PER_W, unroll=2)
    def node_body(n):
        i = w * _PER_W + n

        @pl.when(i < _N)
        def _():
            g = batch_v[pl.ds(i, 16)][0]
            sv = starts_v[pl.ds(g, 16)]
            s = sv[0]
            e = sv[1]
            xi0 = x0[pl.ds(i, 16)][0]
            xi1 = x1[pl.ds(i, 16)][0]
            xi2 = x2[pl.ds(i, 16)][0]
            xi3 = x3[pl.ds(i, 16)][0]
            nch = (e - s + 15) // 16

            def chunk_keys(c):
                idx = c + iot
                d0 = x0[pl.ds(c, 16)] - xi0
                d1 = x1[pl.ds(c, 16)] - xi1
                d2c = x2[pl.ds(c, 16)] - xi2
                dist = d0 * d0 + d1 * d1 + d2c * d2c
                valid = (idx != i) & (idx < e)
                key = jnp.where(valid, dist, _INIT)
                return plsc.sort_key_val(key, idx)

            def chunk_body(t, bc):
                bk, bi = bc
                ck, ci = chunk_keys(s + t * 16)
                return _merge16(bk, bi, ck, ci)

            ck0, ci0 = chunk_keys(s)
            bk, bi = lax.fori_loop(1, nch, chunk_body, (ck0, ci0))

            def _with_fill():
                fill = jnp.where(iot < s, iot,
                                 jnp.where(iot == s, i, e + iot - s - 1))
                fkey = jnp.float32(1e30) * (1.0 + fill.astype(jnp.float32)
                                            * jnp.float32(2.0 ** -10))
                fk, fi = plsc.sort_key_val(fkey, fill)
                return _merge16(bk, bi, fk, fi)

            bk, bi = lax.cond(e - s < 17, _with_fill, lambda: (bk, bi))

            base = n * 16
            for f, (xf, xif) in enumerate(
                    ((x0, xi0), (x1, xi1), (x2, xi2), (x3, xi3))):
                xj = plsc.load_gather(xf, [bi])
                obuf[f, pl.ds(base, 16)] = jnp.broadcast_to(xif, (16,))
                obuf[4 + f, pl.ds(base, 16)] = xj - xif

    for f in range(8):
        pltpu.sync_copy(obuf.at[f], out_hbm.at[f, w])


def _edgeconv_body(w1_ref, b1_ref, w2_ref, b2_ref, w3_ref, b3_ref,
                   planes_ref, h_ref):
    ch = [_rbf(planes_ref[f]) for f in range(8)]
    m1 = []
    for o in range(15):
        acc = ch[0] * w1_ref[0, o]
        for f in range(1, 8):
            acc = acc + ch[f] * w1_ref[f, o]
        m1.append(_rbf(_leaky(acc + b1_ref[o])))
    hpre = None
    for o in range(15):
        acc = m1[0] * w2_ref[0, o]
        for p in range(1, 15):
            acc = acc + m1[p] * w2_ref[p, o]
        m2 = _leaky(acc + b2_ref[o])
        for sh in (1, 2, 4, 8):
            m2 = m2 + pltpu.roll(m2, 128 - sh, 1)
        hq = _rbf(m2)
        contrib = hq * w3_ref[o, 0]
        hpre = contrib if hpre is None else hpre + contrib
    lane = lax.broadcasted_iota(jnp.int32, (_TCR, 128), 1)
    hsel = jnp.where(lane % 16 == 0, hpre, 0.0)
    h_ref[...] = _leaky(jnp.sum(hsel.reshape(_TCR, 8, 16), axis=2)
                        + b3_ref[0])


def _pool_body(h_hbm, starts_hbm, w4b_hbm, out_hbm,
               h_v, starts_v, w4_v, out_v):
    cid = lax.axis_index("c")
    sid = lax.axis_index("s")
    w = sid * 2 + cid

    @pl.when(w < 16)
    def _():
        pltpu.sync_copy(h_hbm, h_v)
        pltpu.sync_copy(starts_hbm, starts_v)
        pltpu.sync_copy(w4b_hbm, w4_v)
        iot = lax.iota(jnp.int32, 16)
        a_vec = jnp.zeros((16,), jnp.float32)
        bm_vec = jnp.zeros((16,), jnp.float32)
        sm_vec = jnp.zeros((16,), jnp.float32)
        for gi in range(16):
            g = w * 16 + gi
            sv = starts_v[pl.ds(g, 16)]
            s = sv[0]
            e = sv[1]
            nch = (e - s + 15) // 16

            def chunk_body(t, carry):
                vmax, vmin, vsum = carry
                c = s + t * 16
                v = h_v[pl.ds(c, 16)]
                m = (c + iot) < e
                vmax = jnp.maximum(vmax, jnp.where(m, v, -_INIT))
                vmin = jnp.minimum(vmin, jnp.where(m, v, _INIT))
                vsum = vsum + jnp.where(m, v, 0.0)
                return vmax, vmin, vsum

            vmax, vmin, vsum = lax.fori_loop(
                0, nch, chunk_body,
                (jnp.full((16,), -_INIT, jnp.float32),
                 jnp.full((16,), _INIT, jnp.float32),
                 jnp.zeros((16,), jnp.float32)))
            lane = iot == gi
            a_vec = jnp.where(lane, jnp.max(vmax), a_vec)
            bm_vec = jnp.where(lane, jnp.min(vmin), bm_vec)
            sm_vec = jnp.where(lane, jnp.sum(vsum), sm_vec)
        s_vec = plsc.load_gather(starts_v, [w * 16 + iot])
        e_vec = plsc.load_gather(starts_v, [w * 16 + 1 + iot])
        cnt = (e_vec - s_vec).astype(jnp.float32)
        pos = cnt > 0
        a = _rbf_sc(jnp.where(pos, a_vec, 0.0))
        bm = _rbf_sc(jnp.where(pos, bm_vec, 0.0))
        sm = _rbf_sc(sm_vec)
        d = _rbf_sc(sm_vec / jnp.maximum(cnt, 1.0))
        w4 = w4_v[...]
        out_v[...] = (a * w4[0] + bm * w4[1] + sm * w4[2]
                      + d * w4[3] + w4[4])
        pltpu.sync_copy(out_v, out_hbm.at[pl.ds(w * 16, 16)])


def _build_sc_kernels():
    mesh = plsc.VectorSubcoreMesh(core_axis_name="c", subcore_axis_name="s",
                                  num_cores=2, num_subcores=16)
    cparams = pltpu.CompilerParams(needs_layout_passes=False)
    knn = pl.kernel(
        _knn_gather_body,
        out_type=jax.ShapeDtypeStruct((8, _NW, _PER_W * _K), jnp.float32),
        mesh=mesh,
        compiler_params=cparams,
        scratch_types=[
            pltpu.VMEM((_XPAD,), jnp.float32),
            pltpu.VMEM((_XPAD,), jnp.float32),
            pltpu.VMEM((_XPAD,), jnp.float32),
            pltpu.VMEM((_XPAD,), jnp.float32),
            pltpu.VMEM((_XPAD,), jnp.int32),
            pltpu.VMEM((_STARTS_PAD,), jnp.int32),
            pltpu.VMEM((8, _PER_W * _K), jnp.float32),
        ],
    )
    pool = pl.kernel(
        _pool_body,
        out_type=jax.ShapeDtypeStruct((_G,), jnp.float32),
        mesh=mesh,
        compiler_params=cparams,
        scratch_types=[
            pltpu.VMEM((_NPAD,), jnp.float32),
            pltpu.VMEM((_STARTS_PAD,), jnp.int32),
            pltpu.VMEM((16,), jnp.float32),
            pltpu.VMEM((16,), jnp.float32),
        ],
    )
    return knn, pool


def kernel(x, batch, edge_index, W1, b1, W2, b2, W3, b3, W4, b4):
    del edge_index
    x = x.astype(jnp.float32)
    batch = batch.astype(jnp.int32)

    xt = jnp.zeros((4, _XPAD), jnp.float32).at[:, :_N].set(x.T)
    batch_p = jnp.zeros((_XPAD,), jnp.int32).at[:_N].set(batch)
    starts = jnp.searchsorted(batch, jnp.arange(_G + 1, dtype=jnp.int32)
                              ).astype(jnp.int32)
    starts = jnp.concatenate(
        [starts, jnp.full((_STARTS_PAD - _G - 1,), _N, jnp.int32)])
    w1r = _rbf(W1.astype(jnp.float32))
    w2r = _rbf(W2.astype(jnp.float32))
    w3r = _rbf(W3.astype(jnp.float32))
    w4b = (jnp.zeros((16,), jnp.float32)
           .at[:4].set(_rbf(W4[:, 0].astype(jnp.float32)))
           .at[4].set(b4[0]))

    knn, pool = _build_sc_kernels()
    planes = knn(xt, batch_p, starts)
    planes = planes.reshape(8, _NK // 128, 128)

    h = pl.pallas_call(
        _edgeconv_body,
        grid=(_NBLK,),
        in_specs=[
            pl.BlockSpec(memory_space=pltpu.SMEM),
            pl.BlockSpec(memory_space=pltpu.SMEM),
            pl.BlockSpec(memory_space=pltpu.SMEM),
            pl.BlockSpec(memory_space=pltpu.SMEM),
            pl.BlockSpec(memory_space=pltpu.SMEM),
            pl.BlockSpec(memory_space=pltpu.SMEM),
            pl.BlockSpec((8, _TCR, 128), lambda j: (0, j, 0)),
        ],
        out_specs=pl.BlockSpec((_TCR, 8), lambda j: (j, 0)),
        out_shape=jax.ShapeDtypeStruct((_NK // 128, 8), jnp.float32),
    )(w1r, b1, w2r, b2, w3r, b3, planes)

    out = pool(h.reshape(_NPAD), starts, w4b)
    return out.reshape(_G, 1)


# speedup vs baseline: 78.5160x; 1.3181x over previous
import jax
import jax.numpy as jnp
from jax import lax
from jax.experimental import pallas as pl
from jax.experimental.pallas import tpu as pltpu
from jax.experimental.pallas import tpu_sc as plsc

_N = 10000
_G = 256
_K = 16
_NW = 32
_PER_W = 320
_NPAD = _NW * _PER_W
_NK = _NPAD * _K
_TCR = 64
_NBLK = _NK // (_TCR * 128)
_STARTS_PAD = 512
_XPAD = _NPAD + 16
_INIT = float(3e38)


def _leaky(t):
    return jnp.where(t > 0, t, 0.01 * t)


def _rbf(t):
    y = lax.bitcast_convert_type(t, jnp.int32)
    r = (y + 0x7FFF + ((y >> 16) & 1)) & jnp.int32(-65536)
    return lax.bitcast_convert_type(r, jnp.float32)


def _rbf_sc(v):
    y = plsc.bitcast(v, jnp.int32)
    r = (y + 0x7FFF + ((y >> 16) & 1)) & jnp.int32(-65536)
    return plsc.bitcast(r, jnp.float32)


def _merge16(ak, ai, bk, bi):
    rbk = jnp.flip(bk, 0)
    rbi = jnp.flip(bi, 0)
    cond = ak <= rbk
    nk = jnp.where(cond, ak, rbk)
    ni = jnp.where(cond, ai, rbi)
    sk, si = plsc.sort_key_val(nk, ni)
    return sk, si


def _knn_gather_body(xt_hbm, batch_hbm, starts_hbm, out_hbm,
                     x0, x1, x2, x3, batch_v, starts_v, obuf):
    cid = lax.axis_index("c")
    sid = lax.axis_index("s")
    w = sid * 2 + cid
    pltpu.sync_copy(xt_hbm.at[0], x0)
    pltpu.sync_copy(xt_hbm.at[1], x1)
    pltpu.sync_copy(xt_hbm.at[2], x2)
    pltpu.sync_copy(xt_hbm.at[3], x3)
    pltpu.sync_copy(batch_hbm, batch_v)
    pltpu.sync_copy(starts_hbm, starts_v)
    iot = lax.iota(jnp.int32, 16)

    @plsc.parallel_loop(0, _PER_W, unroll=2)
    def node_body(n):
        i = w * _PER_W + n

        @pl.when(i < _N)
        def _():
            g = batch_v[pl.ds(i, 16)][0]
            sv = starts_v[pl.ds(g, 16)]
            s = sv[0]
            e = sv[1]
            xi0 = x0[pl.ds(i, 16)][0]
            xi1 = x1[pl.ds(i, 16)][0]
            xi2 = x2[pl.ds(i, 16)][0]
            xi3 = x3[pl.ds(i, 16)][0]
            nch = (e - s + 15) // 16

            def chunk_keys(c):
                idx = c + iot
                d0 = x0[pl.ds(c, 16)] - xi0
                d1 = x1[pl.ds(c, 16)] - xi1
                d2c = x2[pl.ds(c, 16)] - xi2
                dist = d0 * d0 + d1 * d1 + d2c * d2c
                valid = (idx != i) & (idx < e)
                key = jnp.where(valid, dist, _INIT)
                return plsc.sort_key_val(key, idx)

            def chunk_body(t, bc):
                bk, bi = bc
                ck, ci = chunk_keys(s + t * 16)
                return _merge16(bk, bi, ck, ci)

            ck0, ci0 = chunk_keys(s)
            bk, bi = lax.fori_loop(1, nch, chunk_body, (ck0, ci0))

            def _with_fill():
                fill = jnp.where(iot < s, iot,
                                 jnp.where(iot == s, i, e + iot - s - 1))
                fkey = jnp.float32(1e30) * (1.0 + fill.astype(jnp.float32)
                                            * jnp.float32(2.0 ** -10))
                fk, fi = plsc.sort_key_val(fkey, fill)
                return _merge16(bk, bi, fk, fi)

            bk, bi = lax.cond(e - s < 17, _with_fill, lambda: (bk, bi))

            base = n * 16
            for f, (xf, xif) in enumerate(
                    ((x0, xi0), (x1, xi1), (x2, xi2), (x3, xi3))):
                xj = plsc.load_gather(xf, [bi])
                obuf[f, pl.ds(base, 16)] = jnp.broadcast_to(xif, (16,))
                obuf[4 + f, pl.ds(base, 16)] = xj - xif

    for f in range(8):
        pltpu.sync_copy(obuf.at[f], out_hbm.at[f, w])


def _edgeconv_body(w1_ref, b1_ref, w2_ref, b2_ref, w3_ref, b3_ref,
                   planes_ref, h_ref):
    ch = [_rbf(planes_ref[f]) for f in range(8)]
    m1 = []
    for o in range(15):
        acc = ch[0] * w1_ref[0, o]
        for f in range(1, 8):
            acc = acc + ch[f] * w1_ref[f, o]
        m1.append(_rbf(_leaky(acc + b1_ref[o])))
    hpre = None
    for o in range(15):
        acc = m1[0] * w2_ref[0, o]
        for p in range(1, 15):
            acc = acc + m1[p] * w2_ref[p, o]
        m2 = _leaky(acc + b2_ref[o])
        for sh in (1, 2, 4, 8):
            m2 = m2 + pltpu.roll(m2, 128 - sh, 1)
        hq = _rbf(m2)
        contrib = hq * w3_ref[o, 0]
        hpre = contrib if hpre is None else hpre + contrib
    lane = lax.broadcasted_iota(jnp.int32, (_TCR, 128), 1)
    hsel = jnp.where(lane % 16 == 0, hpre, 0.0)
    h_ref[...] = _leaky(jnp.sum(hsel.reshape(_TCR, 8, 16), axis=2)
                        + b3_ref[0])


def _pool_body(h_hbm, starts_hbm, w4b_hbm, out_hbm,
               h_v, starts_v, w4_v, out_v):
    cid = lax.axis_index("c")
    sid = lax.axis_index("s")
    w = sid * 2 + cid

    @pl.when(w < 16)
    def _():
        pltpu.sync_copy(h_hbm, h_v)
        pltpu.sync_copy(starts_hbm, starts_v)
        pltpu.sync_copy(w4b_hbm, w4_v)
        iot = lax.iota(jnp.int32, 16)
        a_vec = jnp.zeros((16,), jnp.float32)
        bm_vec = jnp.zeros((16,), jnp.float32)
        sm_vec = jnp.zeros((16,), jnp.float32)
        for gi in range(16):
            g = w * 16 + gi
            sv = starts_v[pl.ds(g, 16)]
            s = sv[0]
            e = sv[1]
            nch = (e - s + 15) // 16

            def chunk_body(t, carry):
                vmax, vmin, vsum = carry
                c = s + t * 16
                v = h_v[pl.ds(c, 16)]
                m = (c + iot) < e
                vmax = jnp.maximum(vmax, jnp.where(m, v, -_INIT))
                vmin = jnp.minimum(vmin, jnp.where(m, v, _INIT))
                vsum = vsum + jnp.where(m, v, 0.0)
                return vmax, vmin, vsum

            vmax, vmin, vsum = lax.fori_loop(
                0, nch, chunk_body,
                (jnp.full((16,), -_INIT, jnp.float32),
                 jnp.full((16,), _INIT, jnp.float32),
                 jnp.zeros((16,), jnp.float32)))
            lane = iot == gi
            a_vec = jnp.where(lane, jnp.max(vmax), a_vec)
            bm_vec = jnp.where(lane, jnp.min(vmin), bm_vec)
            sm_vec = jnp.where(lane, jnp.sum(vsum), sm_vec)
        s_vec = plsc.load_gather(starts_v, [w * 16 + iot])
        e_vec = plsc.load_gather(starts_v, [w * 16 + 1 + iot])
        cnt = (e_vec - s_vec).astype(jnp.float32)
        pos = cnt > 0
        a = _rbf_sc(jnp.where(pos, a_vec, 0.0))
        bm = _rbf_sc(jnp.where(pos, bm_vec, 0.0))
        sm = _rbf_sc(sm_vec)
        d = _rbf_sc(sm_vec / jnp.maximum(cnt, 1.0))
        w4 = w4_v[...]
        out_v[...] = (a * w4[0] + bm * w4[1] + sm * w4[2]
                      + d * w4[3] + w4[4])
        pltpu.sync_copy(out_v, out_hbm.at[pl.ds(w * 16, 16)])


def _build_sc_kernels():
    mesh = plsc.VectorSubcoreMesh(core_axis_name="c", subcore_axis_name="s",
                                  num_cores=2, num_subcores=16)
    cparams = pltpu.CompilerParams(needs_layout_passes=False)
    knn = pl.kernel(
        _knn_gather_body,
        out_type=jax.ShapeDtypeStruct((8, _NW, _PER_W * _K), jnp.float32),
        mesh=mesh,
        compiler_params=cparams,
        scratch_types=[
            pltpu.VMEM((_XPAD,), jnp.float32),
            pltpu.VMEM((_XPAD,), jnp.float32),
            pltpu.VMEM((_XPAD,), jnp.float32),
            pltpu.VMEM((_XPAD,), jnp.float32),
            pltpu.VMEM((_XPAD,), jnp.int32),
            pltpu.VMEM((_STARTS_PAD,), jnp.int32),
            pltpu.VMEM((8, _PER_W * _K), jnp.float32),
        ],
    )
    pool = pl.kernel(
        _pool_body,
        out_type=jax.ShapeDtypeStruct((_G,), jnp.float32),
        mesh=mesh,
        compiler_params=cparams,
        scratch_types=[
            pltpu.VMEM((_NPAD,), jnp.float32),
            pltpu.VMEM((_STARTS_PAD,), jnp.int32),
            pltpu.VMEM((16,), jnp.float32),
            pltpu.VMEM((16,), jnp.float32),
        ],
    )
    return knn, pool


def kernel(x, batch, edge_index, W1, b1, W2, b2, W3, b3, W4, b4):
    del edge_index
    x = x.astype(jnp.float32)
    batch = batch.astype(jnp.int32)

    xt = jnp.zeros((4, _XPAD), jnp.float32).at[:, :_N].set(x.T)
    batch_p = jnp.zeros((_XPAD,), jnp.int32).at[:_N].set(batch)
    starts = jnp.sum(batch[None, :] < jnp.arange(_G + 1, dtype=jnp.int32)
                     [:, None], axis=1, dtype=jnp.int32)
    starts = jnp.concatenate(
        [starts, jnp.full((_STARTS_PAD - _G - 1,), _N, jnp.int32)])
    w1r = _rbf(W1.astype(jnp.float32))
    w2r = _rbf(W2.astype(jnp.float32))
    w3r = _rbf(W3.astype(jnp.float32))
    w4b = (jnp.zeros((16,), jnp.float32)
           .at[:4].set(_rbf(W4[:, 0].astype(jnp.float32)))
           .at[4].set(b4[0]))

    knn, pool = _build_sc_kernels()
    planes = knn(xt, batch_p, starts)
    planes = planes.reshape(8, _NK // 128, 128)

    h = pl.pallas_call(
        _edgeconv_body,
        grid=(_NBLK,),
        in_specs=[
            pl.BlockSpec(memory_space=pltpu.SMEM),
            pl.BlockSpec(memory_space=pltpu.SMEM),
            pl.BlockSpec(memory_space=pltpu.SMEM),
            pl.BlockSpec(memory_space=pltpu.SMEM),
            pl.BlockSpec(memory_space=pltpu.SMEM),
            pl.BlockSpec(memory_space=pltpu.SMEM),
            pl.BlockSpec((8, _TCR, 128), lambda j: (0, j, 0)),
        ],
        out_specs=pl.BlockSpec((_TCR, 8), lambda j: (j, 0)),
        out_shape=jax.ShapeDtypeStruct((_NK // 128, 8), jnp.float32),
    )(w1r, b1, w2r, b2, w3r, b3, planes)

    out = pool(h.reshape(_NPAD), starts, w4b)
    return out.reshape(_G, 1)


# speedup vs baseline: 84.5564x; 1.0769x over previous
import jax
import jax.numpy as jnp
from jax import lax
from jax.experimental import pallas as pl
from jax.experimental.pallas import tpu as pltpu
from jax.experimental.pallas import tpu_sc as plsc

_N = 10000
_G = 256
_K = 16
_NW = 32
_PER_W = 320
_NPAD = _NW * _PER_W
_NK = _NPAD * _K
_TCR = 32
_NBLK = _NK // (_TCR * 128)
_STARTS_PAD = 512
_XPAD = _NPAD + 16
_INIT = float(3e38)


def _leaky(t):
    return jnp.where(t > 0, t, 0.01 * t)


def _rbf(t):
    y = lax.bitcast_convert_type(t, jnp.int32)
    r = (y + 0x7FFF + ((y >> 16) & 1)) & jnp.int32(-65536)
    return lax.bitcast_convert_type(r, jnp.float32)


def _rbf_sc(v):
    y = plsc.bitcast(v, jnp.int32)
    r = (y + 0x7FFF + ((y >> 16) & 1)) & jnp.int32(-65536)
    return plsc.bitcast(r, jnp.float32)


def _merge16(ak, ai, bk, bi):
    rbk = jnp.flip(bk, 0)
    rbi = jnp.flip(bi, 0)
    cond = ak <= rbk
    nk = jnp.where(cond, ak, rbk)
    ni = jnp.where(cond, ai, rbi)
    sk, si = plsc.sort_key_val(nk, ni)
    return sk, si


def _knn_gather_body(xt_hbm, batch_hbm, starts_hbm, out_hbm,
                     x0, x1, x2, x3, batch_v, starts_v, obuf):
    cid = lax.axis_index("c")
    sid = lax.axis_index("s")
    w = sid * 2 + cid
    pltpu.sync_copy(xt_hbm.at[0], x0)
    pltpu.sync_copy(xt_hbm.at[1], x1)
    pltpu.sync_copy(xt_hbm.at[2], x2)
    pltpu.sync_copy(xt_hbm.at[3], x3)
    pltpu.sync_copy(batch_hbm, batch_v)
    pltpu.sync_copy(starts_hbm, starts_v)
    iot = lax.iota(jnp.int32, 16)

    @plsc.parallel_loop(0, _PER_W, unroll=2)
    def node_body(n):
        i = w * _PER_W + n

        @pl.when(i < _N)
        def _():
            g = batch_v[pl.ds(i, 16)][0]
            sv = starts_v[pl.ds(g, 16)]
            s = sv[0]
            e = sv[1]
            xi0 = x0[pl.ds(i, 16)][0]
            xi1 = x1[pl.ds(i, 16)][0]
            xi2 = x2[pl.ds(i, 16)][0]
            xi3 = x3[pl.ds(i, 16)][0]
            nch = (e - s + 15) // 16

            def chunk_keys(c):
                idx = c + iot
                d0 = x0[pl.ds(c, 16)] - xi0
                d1 = x1[pl.ds(c, 16)] - xi1
                d2c = x2[pl.ds(c, 16)] - xi2
                dist = d0 * d0 + d1 * d1 + d2c * d2c
                valid = (idx != i) & (idx < e)
                key = jnp.where(valid, dist, _INIT)
                return plsc.sort_key_val(key, idx)

            def chunk_body(t, bc):
                bk, bi = bc
                ck, ci = chunk_keys(s + t * 16)
                return _merge16(bk, bi, ck, ci)

            ck0, ci0 = chunk_keys(s)
            bk, bi = lax.fori_loop(1, nch, chunk_body, (ck0, ci0))

            def _with_fill():
                fill = jnp.where(iot < s, iot,
                                 jnp.where(iot == s, i, e + iot - s - 1))
                fkey = jnp.float32(1e30) * (1.0 + fill.astype(jnp.float32)
                                            * jnp.float32(2.0 ** -10))
                fk, fi = plsc.sort_key_val(fkey, fill)
                return _merge16(bk, bi, fk, fi)

            bk, bi = lax.cond(e - s < 17, _with_fill, lambda: (bk, bi))

            base = n * 16
            for f, (xf, xif) in enumerate(
                    ((x0, xi0), (x1, xi1), (x2, xi2), (x3, xi3))):
                xj = plsc.load_gather(xf, [bi])
                obuf[f, pl.ds(base, 16)] = jnp.broadcast_to(xif, (16,))
                obuf[4 + f, pl.ds(base, 16)] = xj - xif

    for f in range(8):
        pltpu.sync_copy(obuf.at[f], out_hbm.at[f, w])


def _edgeconv_body(w1_ref, b1_ref, w2_ref, b2_ref, w3_ref, b3_ref,
                   planes_ref, h_ref):
    ch = [_rbf(planes_ref[f]) for f in range(8)]
    m1 = []
    for o in range(15):
        acc = ch[0] * w1_ref[0, o]
        for f in range(1, 8):
            acc = acc + ch[f] * w1_ref[f, o]
        m1.append(_rbf(_leaky(acc + b1_ref[o])))
    hpre = None
    for o in range(15):
        acc = m1[0] * w2_ref[0, o]
        for p in range(1, 15):
            acc = acc + m1[p] * w2_ref[p, o]
        m2 = _leaky(acc + b2_ref[o])
        for sh in (1, 2, 4, 8):
            m2 = m2 + pltpu.roll(m2, 128 - sh, 1)
        hq = _rbf(m2)
        contrib = hq * w3_ref[o, 0]
        hpre = contrib if hpre is None else hpre + contrib
    lane = lax.broadcasted_iota(jnp.int32, (_TCR, 128), 1)
    hsel = jnp.where(lane % 16 == 0, hpre, 0.0)
    h_ref[...] = _leaky(jnp.sum(hsel.reshape(_TCR, 8, 16), axis=2)
                        + b3_ref[0])


def _pool_body(h_hbm, starts_hbm, w4b_hbm, out_hbm,
               h_v, starts_v, w4_v, out_v):
    cid = lax.axis_index("c")
    sid = lax.axis_index("s")
    w = sid * 2 + cid

    @pl.when(w < 16)
    def _():
        pltpu.sync_copy(h_hbm, h_v)
        pltpu.sync_copy(starts_hbm, starts_v)
        pltpu.sync_copy(w4b_hbm, w4_v)
        iot = lax.iota(jnp.int32, 16)
        a_vec = jnp.zeros((16,), jnp.float32)
        bm_vec = jnp.zeros((16,), jnp.float32)
        sm_vec = jnp.zeros((16,), jnp.float32)
        for gi in range(16):
            g = w * 16 + gi
            sv = starts_v[pl.ds(g, 16)]
            s = sv[0]
            e = sv[1]
            nch = (e - s + 15) // 16

            def chunk_body(t, carry):
                vmax, vmin, vsum = carry
                c = s + t * 16
                v = h_v[pl.ds(c, 16)]
                m = (c + iot) < e
                vmax = jnp.maximum(vmax, jnp.where(m, v, -_INIT))
                vmin = jnp.minimum(vmin, jnp.where(m, v, _INIT))
                vsum = vsum + jnp.where(m, v, 0.0)
                return vmax, vmin, vsum

            vmax, vmin, vsum = lax.fori_loop(
                0, nch, chunk_body,
                (jnp.full((16,), -_INIT, jnp.float32),
                 jnp.full((16,), _INIT, jnp.float32),
                 jnp.zeros((16,), jnp.float32)))
            lane = iot == gi
            a_vec = jnp.where(lane, jnp.max(vmax), a_vec)
            bm_vec = jnp.where(lane, jnp.min(vmin), bm_vec)
            sm_vec = jnp.where(lane, jnp.sum(vsum), sm_vec)
        s_vec = plsc.load_gather(starts_v, [w * 16 + iot])
        e_vec = plsc.load_gather(starts_v, [w * 16 + 1 + iot])
        cnt = (e_vec - s_vec).astype(jnp.float32)
        pos = cnt > 0
        a = _rbf_sc(jnp.where(pos, a_vec, 0.0))
        bm = _rbf_sc(jnp.where(pos, bm_vec, 0.0))
        sm = _rbf_sc(sm_vec)
        d = _rbf_sc(sm_vec / jnp.maximum(cnt, 1.0))
        w4 = w4_v[...]
        out_v[...] = (a * w4[0] + bm * w4[1] + sm * w4[2]
                      + d * w4[3] + w4[4])
        pltpu.sync_copy(out_v, out_hbm.at[pl.ds(w * 16, 16)])


def _build_sc_kernels():
    mesh = plsc.VectorSubcoreMesh(core_axis_name="c", subcore_axis_name="s",
                                  num_cores=2, num_subcores=16)
    cparams = pltpu.CompilerParams(needs_layout_passes=False)
    knn = pl.kernel(
        _knn_gather_body,
        out_type=jax.ShapeDtypeStruct((8, _NW, _PER_W * _K), jnp.float32),
        mesh=mesh,
        compiler_params=cparams,
        scratch_types=[
            pltpu.VMEM((_XPAD,), jnp.float32),
            pltpu.VMEM((_XPAD,), jnp.float32),
            pltpu.VMEM((_XPAD,), jnp.float32),
            pltpu.VMEM((_XPAD,), jnp.float32),
            pltpu.VMEM((_XPAD,), jnp.int32),
            pltpu.VMEM((_STARTS_PAD,), jnp.int32),
            pltpu.VMEM((8, _PER_W * _K), jnp.float32),
        ],
    )
    pool = pl.kernel(
        _pool_body,
        out_type=jax.ShapeDtypeStruct((_G,), jnp.float32),
        mesh=mesh,
        compiler_params=cparams,
        scratch_types=[
            pltpu.VMEM((_NPAD,), jnp.float32),
            pltpu.VMEM((_STARTS_PAD,), jnp.int32),
            pltpu.VMEM((16,), jnp.float32),
            pltpu.VMEM((16,), jnp.float32),
        ],
    )
    return knn, pool


def kernel(x, batch, edge_index, W1, b1, W2, b2, W3, b3, W4, b4):
    del edge_index
    x = x.astype(jnp.float32)
    batch = batch.astype(jnp.int32)

    xt = jnp.zeros((4, _XPAD), jnp.float32).at[:, :_N].set(x.T)
    batch_p = jnp.zeros((_XPAD,), jnp.int32).at[:_N].set(batch)
    starts = jnp.sum(batch[None, :] < jnp.arange(_G + 1, dtype=jnp.int32)
                     [:, None], axis=1, dtype=jnp.int32)
    starts = jnp.concatenate(
        [starts, jnp.full((_STARTS_PAD - _G - 1,), _N, jnp.int32)])
    w1r = _rbf(W1.astype(jnp.float32))
    w2r = _rbf(W2.astype(jnp.float32))
    w3r = _rbf(W3.astype(jnp.float32))
    w4b = (jnp.zeros((16,), jnp.float32)
           .at[:4].set(_rbf(W4[:, 0].astype(jnp.float32)))
           .at[4].set(b4[0]))

    knn, pool = _build_sc_kernels()
    planes = knn(xt, batch_p, starts)
    planes = planes.reshape(8, _NK // 128, 128)

    h = pl.pallas_call(
        _edgeconv_body,
        grid=(_NBLK,),
        in_specs=[
            pl.BlockSpec(memory_space=pltpu.SMEM),
            pl.BlockSpec(memory_space=pltpu.SMEM),
            pl.BlockSpec(memory_space=pltpu.SMEM),
            pl.BlockSpec(memory_space=pltpu.SMEM),
            pl.BlockSpec(memory_space=pltpu.SMEM),
            pl.BlockSpec(memory_space=pltpu.SMEM),
            pl.BlockSpec((8, _TCR, 128), lambda j: (0, j, 0)),
        ],
        out_specs=pl.BlockSpec((_TCR, 8), lambda j: (j, 0)),
        out_shape=jax.ShapeDtypeStruct((_NK // 128, 8), jnp.float32),
    )(w1r, b1, w2r, b2, w3r, b3, planes)

    out = pool(h.reshape(_NPAD), starts, w4b)
    return out.reshape(_G, 1)
